# baseline (speedup 1.0000x reference)
import jax
import jax.numpy as jnp
from jax import lax
from jax.experimental import pallas as pl
from jax.experimental.pallas import tpu as pltpu
from jax.experimental.pallas import tpu_sc as plsc

N = 10000
F = 64
KK = 8
POS_PER_ROW = KK * KK
ROWS_PER_GROUP = 8
GROUPS = N // ROWS_PER_GROUP
POS_PER_GROUP = ROWS_PER_GROUP * POS_PER_ROW
SRC_PER_GROUP = POS_PER_GROUP * 2
OUT_ROW = 2 * F
OUT_ROWS_PER_GROUP = POS_PER_GROUP * F // OUT_ROW
NUM_WORKERS = 32
MAX_GPW = (GROUPS + NUM_WORKERS - 1) // NUM_WORKERS
NBUF = 2
LANES = 16


def _sc_body(dst_hbm, src_hbm, e1_hbm, e2_hbm, e3_hbm, out_hbm,
             dst_v, e1_v, e2_v, e3_v, t2_v, *bufs_and_sems):
    cid = lax.axis_index("c")
    sid = lax.axis_index("s")
    wid = sid * 2 + cid

    src_bufs = bufs_and_sems[:NBUF]
    out_bufs = bufs_and_sems[NBUF:2 * NBUF]
    src_sems = bufs_and_sems[2 * NBUF:3 * NBUF]
    out_sems = bufs_and_sems[3 * NBUF:]

    def fire_src(gi, b):
        g = wid + gi * NUM_WORKERS

        @pl.when(g < GROUPS)
        def _():
            pltpu.async_copy(
                src_hbm.at[pl.ds(g * SRC_PER_GROUP, SRC_PER_GROUP)],
                src_bufs[b],
                src_sems[b])

    pltpu.sync_copy(dst_hbm, dst_v.at[pl.ds(0, N)])
    pltpu.sync_copy(e1_hbm, e1_v)
    pltpu.sync_copy(e2_hbm, e2_v)
    pltpu.sync_copy(e3_hbm, e3_v)

    for b in range(NBUF):
        fire_src(jnp.int32(b), b)

    for r in range(8):
        b2, b1, b0 = (r >> 2) & 1, (r >> 1) & 1, r & 1
        for c in range(F // LANES):
            off = c * LANES
            t2_v[pl.ds(r * F + off, LANES)] = (
                e2_v[pl.ds(b2 * F + off, LANES)]
                + e1_v[pl.ds(b1 * F + off, LANES)]
                + e3_v[pl.ds(b0 * F + off, LANES)])

    def ring_body(p, carry):
        for b in range(NBUF):
            gi = p * NBUF + b
            g = wid + gi * NUM_WORKERS

            @pl.when(g < GROUPS)
            def _():
                src_v = src_bufs[b]
                out_v = out_bufs[b]
                pltpu.make_async_copy(
                    src_hbm.at[pl.ds(0, SRC_PER_GROUP)],
                    src_v,
                    src_sems[b]).wait()
                @pl.when(gi >= NBUF)
                def _():
                    pltpu.make_async_copy(
                        out_v,
                        out_hbm.at[pl.ds(0, OUT_ROWS_PER_GROUP)],
                        out_sems[b]).wait()

                @plsc.parallel_loop(0, POS_PER_GROUP // 8, unroll=2)
                def chunk_body(cp):
                    pairv = src_v[pl.ds(cp * 16, 16)]
                    zav = dst_v[pl.ds(g * ROWS_PER_GROUP + (cp >> 3), 16)]
                    za = zav[0]
                    for j in range(8):
                        zb = pairv[2 * j]
                        zc = pairv[2 * j + 1]
                        idx = (((za == zb).astype(jnp.int32) << 2)
                               | ((za == zc).astype(jnp.int32) << 1)
                               | (zb == zc).astype(jnp.int32))
                        tbase = idx << 6
                        orow = cp * 4 + (j >> 1)
                        ocol = (j & 1) * F
                        for fb in range(0, F, LANES):
                            out_v[orow, pl.ds(ocol + fb, LANES)] = (
                                t2_v[pl.ds(tbase + fb, LANES)])

                pltpu.async_copy(
                    out_v,
                    out_hbm.at[pl.ds(g * OUT_ROWS_PER_GROUP,
                                     OUT_ROWS_PER_GROUP)],
                    out_sems[b])
                fire_src(gi + NBUF, b)

        return carry

    lax.fori_loop(0, (MAX_GPW + NBUF - 1) // NBUF, ring_body, 0)

    for b in range(NBUF):
        pltpu.make_async_copy(
            out_bufs[b],
            out_hbm.at[pl.ds(0, OUT_ROWS_PER_GROUP)],
            out_sems[b]).wait()


@jax.jit
def _run(dst_adj, src_flat, e1f, e2f, e3f):
    mesh = plsc.VectorSubcoreMesh(core_axis_name="c", subcore_axis_name="s",
                                  num_cores=2, num_subcores=16)
    f = pl.kernel(
        _sc_body,
        out_type=jax.ShapeDtypeStruct((N * POS_PER_ROW * F // OUT_ROW,
                                       OUT_ROW), jnp.float32),
        mesh=mesh,
        compiler_params=pltpu.CompilerParams(needs_layout_passes=False),
        scratch_types=[
            pltpu.VMEM((N + 16,), jnp.int32),
            pltpu.VMEM((2 * F,), jnp.float32),
            pltpu.VMEM((2 * F,), jnp.float32),
            pltpu.VMEM((2 * F,), jnp.float32),
            pltpu.VMEM((8 * F,), jnp.float32),
        ] + [pltpu.VMEM((SRC_PER_GROUP,), jnp.int32)] * NBUF
          + [pltpu.VMEM((OUT_ROWS_PER_GROUP, OUT_ROW), jnp.float32)] * NBUF
          + [pltpu.SemaphoreType.DMA] * (2 * NBUF),
    )
    return f(dst_adj, src_flat, e1f, e2f, e3f)


def kernel(dst_z, src_z, k, e1_weight, e2_weight, e3_weight):
    kk = src_z.shape[1]
    dst_adj = (dst_z + (jnp.asarray(k, jnp.int32) - kk)).astype(jnp.int32)
    src_flat = src_z.reshape(-1)
    out_rows = _run(dst_adj, src_flat,
                    e1_weight.reshape(-1), e2_weight.reshape(-1),
                    e3_weight.reshape(-1))
    return out_rows.reshape(N, KK, KK, F)
